# Initial kernel scaffold; baseline (speedup 1.0000x reference)
import jax
import jax.numpy as jnp
from jax.experimental import pallas as pl


def kernel(img, embeddings, edge_index, W0, b0, W1, b1, W2, b2, W3, b3, Wg1, bg1, Wg2, bg2):
    raise NotImplementedError("write your pallas kernel here")


# speedup vs baseline: 5.5635x; 5.5635x over previous
import functools

import jax
import jax.numpy as jnp
from jax import lax
from jax.experimental import pallas as pl
from jax.experimental.pallas import tpu as pltpu
from jax.experimental.pallas import tpu_sc as plsc

DISP = 1100
N_NODES = 31100
NP = 31104
DP = 320
DC = 32
NPASS = DP // DC
NT = 32
EB = 128
NB_E = 52
EP = NT * NB_E * EB
ROWS_PT = NP // 16

_MESH = plsc.VectorSubcoreMesh(core_axis_name="c", subcore_axis_name="s")
_SC_PARAMS = pltpu.CompilerParams(use_tc_tiling_on_sc=False)


def _sc_deg_body(rowb_hbm, ones_hbm, zeros_hbm, out_hbm,
                 rowbuf, onesb, zbuf, acc):
    c = lax.axis_index("c")
    s = lax.axis_index("s")
    wid = c * 16 + s
    pltpu.sync_copy(rowb_hbm.at[wid], rowbuf)
    pltpu.sync_copy(ones_hbm, onesb)
    pltpu.sync_copy(zeros_hbm, zbuf)
    r0 = s * ROWS_PT
    for off in range(0, ROWS_PT - 128 + 1, 128):
        pltpu.sync_copy(zbuf, acc.at[pl.ds(r0 + off, 128)])
    rem = ROWS_PT % 128
    if rem:
        pltpu.sync_copy(zbuf.at[pl.ds(0, rem)],
                        acc.at[pl.ds(r0 + ROWS_PT - rem, rem)])
    plsc.subcore_barrier()
    for j in range(NB_E):
        pltpu.sync_copy(onesb, acc.at[rowbuf.at[j]], add=True)
    plsc.subcore_barrier()
    for off in range(0, ROWS_PT - 128 + 1, 128):
        pltpu.sync_copy(acc.at[pl.ds(r0 + off, 128)],
                        out_hbm.at[c, pl.ds(r0 + off, 128)])
    if rem:
        pltpu.sync_copy(acc.at[pl.ds(r0 + ROWS_PT - rem, rem)],
                        out_hbm.at[c, pl.ds(r0 + ROWS_PT - rem, rem)])


_sc_deg = pl.kernel(
    _sc_deg_body,
    out_type=jax.ShapeDtypeStruct((2, NP, 16), jnp.float32),
    mesh=_MESH,
    scratch_types=[
        pltpu.VMEM((NB_E, EB), jnp.int32),
        pltpu.VMEM((EB, 16), jnp.float32),
        pltpu.VMEM((128, 16), jnp.float32),
        pltpu.VMEM_SHARED((NP, 16), jnp.float32),
    ],
    compiler_params=_SC_PARAMS,
)


def _sc_spmm_body(xs_hbm, rowb_hbm, colb_hbm, zeros_hbm, out_hbm,
                  rowbuf, colbuf, gbuf0, gbuf1, zbuf, acc, sem0, sem1):
    c = lax.axis_index("c")
    s = lax.axis_index("s")
    wid = c * 16 + s
    pltpu.sync_copy(rowb_hbm.at[wid], rowbuf)
    pltpu.sync_copy(colb_hbm.at[wid], colbuf)
    pltpu.sync_copy(zeros_hbm, zbuf)
    r0 = s * ROWS_PT
    rem = ROWS_PT % 128

    def one_pass(p, carry):
        for off in range(0, ROWS_PT - 128 + 1, 128):
            pltpu.sync_copy(zbuf, acc.at[pl.ds(r0 + off, 128)])
        if rem:
            pltpu.sync_copy(zbuf.at[pl.ds(0, rem)],
                            acc.at[pl.ds(r0 + ROWS_PT - rem, rem)])
        plsc.subcore_barrier()
        xs_p = xs_hbm.at[p]
        gb = (gbuf0, gbuf1)
        sems = (sem0, sem1)
        descs = [None, None]
        for j in range(NB_E):
            b = j & 1
            descs[b] = pltpu.async_copy(xs_p.at[colbuf.at[j]], gb[b], sems[b])
            if j > 0:
                descs[1 - b].wait()
                pltpu.sync_copy(gb[1 - b], acc.at[rowbuf.at[j - 1]], add=True)
        descs[(NB_E - 1) & 1].wait()
        pltpu.sync_copy(gb[(NB_E - 1) & 1],
                        acc.at[rowbuf.at[NB_E - 1]], add=True)
        plsc.subcore_barrier()
        for off in range(0, ROWS_PT - 128 + 1, 128):
            pltpu.sync_copy(acc.at[pl.ds(r0 + off, 128)],
                            out_hbm.at[c, p, pl.ds(r0 + off, 128)])
        if rem:
            pltpu.sync_copy(acc.at[pl.ds(r0 + ROWS_PT - rem, rem)],
                            out_hbm.at[c, p, pl.ds(r0 + ROWS_PT - rem, rem)])
        plsc.subcore_barrier()
        return carry

    lax.fori_loop(0, NPASS, one_pass, 0)


_sc_spmm = pl.kernel(
    _sc_spmm_body,
    out_type=jax.ShapeDtypeStruct((2, NPASS, NP, DC), jnp.float32),
    mesh=_MESH,
    scratch_types=[
        pltpu.VMEM((NB_E, EB), jnp.int32),
        pltpu.VMEM((NB_E, EB), jnp.int32),
        pltpu.VMEM((EB, DC), jnp.float32),
        pltpu.VMEM((EB, DC), jnp.float32),
        pltpu.VMEM((128, DC), jnp.float32),
        pltpu.VMEM_SHARED((NP, DC), jnp.float32),
        pltpu.SemaphoreType.DMA,
        pltpu.SemaphoreType.DMA,
    ],
    compiler_params=_SC_PARAMS,
)


def _dinv_of(degp_block):
    deg = jnp.sum(degp_block, axis=(0, 2)) * (1.0 / 16.0)
    return 1.0 / jnp.sqrt(jnp.maximum(deg, 1.0))


def _prescale_body(degp_ref, emb_ref, xs_ref):
    dinv = _dinv_of(degp_ref[...])
    xs_ref[...] = emb_ref[...] * dinv[:, None]


def _tc_prescale(degp, embp):
    bm = 3888
    grid = (NP // bm,)
    return pl.pallas_call(
        _prescale_body,
        grid=grid,
        in_specs=[
            pl.BlockSpec((2, bm, 16), lambda i: (0, i, 0)),
            pl.BlockSpec((bm, DP), lambda i: (i, 0)),
        ],
        out_specs=pl.BlockSpec((bm, DP), lambda i: (i, 0)),
        out_shape=jax.ShapeDtypeStruct((NP, DP), jnp.float32),
    )(degp, embp)


def _gcn_mm_body(p0_ref, p1_ref, degp_ref, wg1_ref, bg1_ref, wg2_ref,
                 out_ref):
    dinv = _dinv_of(degp_ref[...])
    acc = jnp.zeros((p0_ref.shape[1], wg1_ref.shape[1]), jnp.float32)
    for p in range(NPASS):
        h1p = (p0_ref[p] + p1_ref[p]) * dinv[:, None]
        acc = acc + jnp.dot(h1p, wg1_ref[pl.ds(p * DC, DC), :],
                            preferred_element_type=jnp.float32)
    g = jnp.maximum(acc + bg1_ref[...][None, :], 0.0)
    g2 = jnp.dot(g, wg2_ref[...], preferred_element_type=jnp.float32)
    out_ref[...] = g2 * dinv[:, None]


def _tc_gcn_mm(p0, p1, degp, wg1p, bg1, wg2p):
    bm = 432
    grid = (NP // bm,)
    return pl.pallas_call(
        _gcn_mm_body,
        grid=grid,
        in_specs=[
            pl.BlockSpec((NPASS, bm, DC), lambda i: (0, i, 0)),
            pl.BlockSpec((NPASS, bm, DC), lambda i: (0, i, 0)),
            pl.BlockSpec((2, bm, 16), lambda i: (0, i, 0)),
            pl.BlockSpec((DP, 4096), lambda i: (0, 0)),
            pl.BlockSpec((4096,), lambda i: (0,)),
            pl.BlockSpec((4096, DP), lambda i: (0, 0)),
        ],
        out_specs=pl.BlockSpec((bm, DP), lambda i: (i, 0)),
        out_shape=jax.ShapeDtypeStruct((NP, DP), jnp.float32),
    )(p0, p1, degp, wg1p, bg1, wg2p)


def _mlp_body(img_ref, w0_ref, b0_ref, w1_ref, b1_ref, w2_ref, b2_ref,
              w3_ref, b3_ref, out_ref):
    h = jnp.maximum(jnp.dot(img_ref[...], w0_ref[...],
                            preferred_element_type=jnp.float32)
                    + b0_ref[...][None, :], 0.0)
    h = jnp.maximum(jnp.dot(h, w1_ref[...],
                            preferred_element_type=jnp.float32)
                    + b1_ref[...][None, :], 0.0)
    h = jnp.maximum(jnp.dot(h, w2_ref[...],
                            preferred_element_type=jnp.float32)
                    + b2_ref[...][None, :], 0.0)
    h = jnp.dot(h, w3_ref[...], preferred_element_type=jnp.float32) \
        + b3_ref[...][None, :]
    nrm = jnp.sqrt(jnp.sum(h * h, axis=1, keepdims=True))
    out_ref[...] = h / (nrm + 1e-8)


def _tc_mlp(img, w0, b0, w1, b1, w2, b2, w3p, b3p):
    B = img.shape[0]
    bm = 256
    grid = (B // bm,)
    full = lambda *s: pl.BlockSpec(s, lambda i: tuple(0 for _ in s))
    return pl.pallas_call(
        _mlp_body,
        grid=grid,
        in_specs=[
            pl.BlockSpec((bm, 512), lambda i: (i, 0)),
            full(512, 768), full(768,),
            full(768, 1024), full(1024,),
            full(1024, 1200), full(1200,),
            full(1200, DP), full(DP,),
        ],
        out_specs=pl.BlockSpec((bm, DP), lambda i: (i, 0)),
        out_shape=jax.ShapeDtypeStruct((B, DP), jnp.float32),
    )(img, w0, b0, w1, b1, w2, b2, w3p, b3p)


def _score_body(a_ref, b_ref, degp_ref, bg2_ref, imgf_ref, out_ref):
    dinv = _dinv_of(degp_ref[...])
    acc = jnp.zeros((imgf_ref.shape[0], a_ref.shape[1]), jnp.float32)
    for p in range(NPASS):
        pair = (a_ref[p] + b_ref[p]) * dinv[:, None] \
            + bg2_ref[pl.ds(p * DC, DC)][None, :]
        acc = acc + lax.dot_general(
            imgf_ref[:, pl.ds(p * DC, DC)], pair,
            (((1,), (1,)), ((), ())),
            preferred_element_type=jnp.float32)
    out_ref[...] = acc


def _tc_score(o2a, o2b, degp_pair, bg2p, imgf):
    bn = 1280
    npair = o2a.shape[1]
    grid = (npair // bn,)
    B = imgf.shape[0]
    return pl.pallas_call(
        _score_body,
        grid=grid,
        in_specs=[
            pl.BlockSpec((NPASS, bn, DC), lambda j: (0, j, 0)),
            pl.BlockSpec((NPASS, bn, DC), lambda j: (0, j, 0)),
            pl.BlockSpec((2, bn, 16), lambda j: (0, j, 0)),
            pl.BlockSpec((DP,), lambda j: (0,)),
            pl.BlockSpec((B, DP), lambda j: (0, 0)),
        ],
        out_specs=pl.BlockSpec((B, bn), lambda j: (0, j)),
        out_shape=jax.ShapeDtypeStruct((B, npair), jnp.float32),
    )(o2a, o2b, degp_pair, bg2p, imgf)


@jax.jit
def kernel(img, embeddings, edge_index, W0, b0, W1, b1, W2, b2, W3, b3,
           Wg1, bg1, Wg2, bg2):
    f32 = jnp.float32
    row = edge_index[0].astype(jnp.int32)
    col = edge_index[1].astype(jnp.int32)
    E = row.shape[0]
    row = jnp.pad(row, (0, EP - E), constant_values=N_NODES)
    col = jnp.pad(col, (0, EP - E), constant_values=N_NODES)
    rowb = row.reshape(NT, NB_E, EB)
    colb = col.reshape(NT, NB_E, EB)
    embp = jnp.pad(embeddings.astype(f32),
                   ((0, NP - N_NODES), (0, DP - embeddings.shape[1])))
    wg1p = jnp.pad(Wg1.astype(f32), ((0, DP - Wg1.shape[0]), (0, 0)))
    wg2p = jnp.pad(Wg2.astype(f32), ((0, 0), (0, DP - Wg2.shape[1])))
    bg2p = jnp.pad(bg2.astype(f32), (0, DP - bg2.shape[0]))
    w3p = jnp.pad(W3.astype(f32), ((0, 0), (0, DP - W3.shape[1])))
    b3p = jnp.pad(b3.astype(f32), (0, DP - b3.shape[0]))
    ones16 = jnp.ones((EB, 16), f32)
    zeros16 = jnp.zeros((128, 16), f32)
    zeros64 = jnp.zeros((128, DC), f32)

    degp = _sc_deg(rowb, ones16, zeros16)
    xs = _tc_prescale(degp, embp)
    xs_t = jnp.transpose(xs.reshape(NP, NPASS, DC), (1, 0, 2))

    o1 = _sc_spmm(xs_t, rowb, colb, zeros64)
    g2s = _tc_gcn_mm(o1[0], o1[1], degp, wg1p, bg1.astype(f32), wg2p)
    g2s_t = jnp.transpose(g2s.reshape(NP, NPASS, DC), (1, 0, 2))

    o2 = _sc_spmm(g2s_t, rowb, colb, zeros64)

    imgf = _tc_mlp(img.astype(f32), W0.astype(f32), b0.astype(f32),
                   W1.astype(f32), b1.astype(f32), W2.astype(f32),
                   b2.astype(f32), w3p, b3p)
    npair = N_NODES - DISP
    npair_p = 30720
    pad = npair_p - npair
    o2a = jnp.pad(o2[0, :, DISP:N_NODES, :], ((0, 0), (0, pad), (0, 0)))
    o2b = jnp.pad(o2[1, :, DISP:N_NODES, :], ((0, 0), (0, pad), (0, 0)))
    degp_pair = jnp.pad(degp[:, DISP:N_NODES, :],
                        ((0, 0), (0, pad), (0, 0)))
    score = _tc_score(o2a, o2b, degp_pair, bg2p, imgf)
    return score[:, :npair]


# speedup vs baseline: 6.4085x; 1.1519x over previous
import functools

import jax
import jax.numpy as jnp
from jax import lax
from jax.experimental import pallas as pl
from jax.experimental.pallas import tpu as pltpu
from jax.experimental.pallas import tpu_sc as plsc

DISP = 1100
N_NODES = 31100
NP = 31104
DP = 320
DC = 32
NPASS = DP // DC
NT = 32
EB = 128
NB_E = 52
EP = NT * NB_E * EB
ROWS_PT = NP // 16

_MESH = plsc.VectorSubcoreMesh(core_axis_name="c", subcore_axis_name="s")
_SC_PARAMS = pltpu.CompilerParams(use_tc_tiling_on_sc=False)


def _sc_deg_body(rowb_hbm, ones_hbm, zeros_hbm, out_hbm,
                 rowbuf, onesb, zbuf, acc):
    c = lax.axis_index("c")
    s = lax.axis_index("s")
    wid = c * 16 + s
    pltpu.sync_copy(rowb_hbm.at[wid], rowbuf)
    pltpu.sync_copy(ones_hbm, onesb)
    pltpu.sync_copy(zeros_hbm, zbuf)
    r0 = s * ROWS_PT
    for off in range(0, ROWS_PT - 128 + 1, 128):
        pltpu.sync_copy(zbuf, acc.at[pl.ds(r0 + off, 128)])
    rem = ROWS_PT % 128
    if rem:
        pltpu.sync_copy(zbuf.at[pl.ds(0, rem)],
                        acc.at[pl.ds(r0 + ROWS_PT - rem, rem)])
    plsc.subcore_barrier()
    for j in range(NB_E):
        pltpu.sync_copy(onesb, acc.at[rowbuf.at[j]], add=True)
    plsc.subcore_barrier()
    for off in range(0, ROWS_PT - 128 + 1, 128):
        pltpu.sync_copy(acc.at[pl.ds(r0 + off, 128)],
                        out_hbm.at[c, pl.ds(r0 + off, 128)])
    if rem:
        pltpu.sync_copy(acc.at[pl.ds(r0 + ROWS_PT - rem, rem)],
                        out_hbm.at[c, pl.ds(r0 + ROWS_PT - rem, rem)])


_sc_deg = pl.kernel(
    _sc_deg_body,
    out_type=jax.ShapeDtypeStruct((2, NP, 16), jnp.float32),
    mesh=_MESH,
    scratch_types=[
        pltpu.VMEM((NB_E, EB), jnp.int32),
        pltpu.VMEM((EB, 16), jnp.float32),
        pltpu.VMEM((128, 16), jnp.float32),
        pltpu.VMEM_SHARED((NP, 16), jnp.float32),
    ],
    compiler_params=_SC_PARAMS,
)


def _sc_spmm_body(xs_hbm, rowb_hbm, colb_hbm, zeros_hbm, out_hbm,
                  rowbuf, colbuf, gbuf0, gbuf1, zbuf, acc, sem0, sem1):
    c = lax.axis_index("c")
    s = lax.axis_index("s")
    wid = c * 16 + s
    pltpu.sync_copy(rowb_hbm.at[wid], rowbuf)
    pltpu.sync_copy(colb_hbm.at[wid], colbuf)
    pltpu.sync_copy(zeros_hbm, zbuf)
    r0 = s * ROWS_PT
    rem = ROWS_PT % 128

    def one_pass(p, carry):
        for off in range(0, ROWS_PT - 128 + 1, 128):
            pltpu.sync_copy(zbuf, acc.at[pl.ds(r0 + off, 128)])
        if rem:
            pltpu.sync_copy(zbuf.at[pl.ds(0, rem)],
                            acc.at[pl.ds(r0 + ROWS_PT - rem, rem)])
        plsc.subcore_barrier()
        xs_p = xs_hbm.at[p]
        gb = (gbuf0, gbuf1)
        sems = (sem0, sem1)
        descs = [None, None]
        for j in range(NB_E):
            b = j & 1
            descs[b] = pltpu.async_copy(xs_p.at[colbuf.at[j]], gb[b], sems[b])
            if j > 0:
                descs[1 - b].wait()
                pltpu.sync_copy(gb[1 - b], acc.at[rowbuf.at[j - 1]], add=True)
        descs[(NB_E - 1) & 1].wait()
        pltpu.sync_copy(gb[(NB_E - 1) & 1],
                        acc.at[rowbuf.at[NB_E - 1]], add=True)
        plsc.subcore_barrier()
        for off in range(0, ROWS_PT - 128 + 1, 128):
            pltpu.sync_copy(acc.at[pl.ds(r0 + off, 128)],
                            out_hbm.at[c, p, pl.ds(r0 + off, 128)])
        if rem:
            pltpu.sync_copy(acc.at[pl.ds(r0 + ROWS_PT - rem, rem)],
                            out_hbm.at[c, p, pl.ds(r0 + ROWS_PT - rem, rem)])
        plsc.subcore_barrier()
        return carry

    lax.fori_loop(0, NPASS, one_pass, 0)


_sc_spmm = pl.kernel(
    _sc_spmm_body,
    out_type=jax.ShapeDtypeStruct((2, NPASS, NP, DC), jnp.float32),
    mesh=_MESH,
    scratch_types=[
        pltpu.VMEM((NB_E, EB), jnp.int32),
        pltpu.VMEM((NB_E, EB), jnp.int32),
        pltpu.VMEM((EB, DC), jnp.float32),
        pltpu.VMEM((EB, DC), jnp.float32),
        pltpu.VMEM((128, DC), jnp.float32),
        pltpu.VMEM_SHARED((NP, DC), jnp.float32),
        pltpu.SemaphoreType.DMA,
        pltpu.SemaphoreType.DMA,
    ],
    compiler_params=_SC_PARAMS,
)


def _dinv_of(degp_block):
    deg = jnp.sum(degp_block, axis=(0, 2)) * (1.0 / 16.0)
    return 1.0 / jnp.sqrt(jnp.maximum(deg, 1.0))


def _prescale_body(degp_ref, emb_ref, xs_ref):
    dinv = _dinv_of(degp_ref[...])
    xs_ref[...] = emb_ref[...] * dinv[:, None]


def _tc_prescale(degp, embp):
    bm = 3888
    grid = (NP // bm,)
    return pl.pallas_call(
        _prescale_body,
        grid=grid,
        in_specs=[
            pl.BlockSpec((2, bm, 16), lambda i: (0, i, 0)),
            pl.BlockSpec((bm, DP), lambda i: (i, 0)),
        ],
        out_specs=pl.BlockSpec((bm, DP), lambda i: (i, 0)),
        out_shape=jax.ShapeDtypeStruct((NP, DP), jnp.float32),
    )(degp, embp)


def _gcn_mm_body(p0_ref, p1_ref, degp_ref, wg1_ref, bg1_ref, wg2_ref,
                 out_ref):
    dinv = _dinv_of(degp_ref[...])
    h1 = jnp.concatenate(
        [p0_ref[p] + p1_ref[p] for p in range(NPASS)], axis=1
    ) * dinv[:, None]
    acc = jnp.dot(h1, wg1_ref[...], preferred_element_type=jnp.float32)
    g = jnp.maximum(acc + bg1_ref[...][None, :], 0.0)
    g2 = jnp.dot(g, wg2_ref[...], preferred_element_type=jnp.float32)
    out_ref[...] = g2 * dinv[:, None]


def _tc_gcn_mm(p0, p1, degp, wg1p, bg1, wg2p):
    bm = 432
    grid = (NP // bm,)
    return pl.pallas_call(
        _gcn_mm_body,
        grid=grid,
        in_specs=[
            pl.BlockSpec((NPASS, bm, DC), lambda i: (0, i, 0)),
            pl.BlockSpec((NPASS, bm, DC), lambda i: (0, i, 0)),
            pl.BlockSpec((2, bm, 16), lambda i: (0, i, 0)),
            pl.BlockSpec((DP, 4096), lambda i: (0, 0)),
            pl.BlockSpec((4096,), lambda i: (0,)),
            pl.BlockSpec((4096, DP), lambda i: (0, 0)),
        ],
        out_specs=pl.BlockSpec((bm, DP), lambda i: (i, 0)),
        out_shape=jax.ShapeDtypeStruct((NP, DP), jnp.float32),
    )(p0, p1, degp, wg1p, bg1, wg2p)


def _mlp_body(img_ref, w0_ref, b0_ref, w1_ref, b1_ref, w2_ref, b2_ref,
              w3_ref, b3_ref, out_ref):
    h = jnp.maximum(jnp.dot(img_ref[...], w0_ref[...],
                            preferred_element_type=jnp.float32)
                    + b0_ref[...][None, :], 0.0)
    h = jnp.maximum(jnp.dot(h, w1_ref[...],
                            preferred_element_type=jnp.float32)
                    + b1_ref[...][None, :], 0.0)
    h = jnp.maximum(jnp.dot(h, w2_ref[...],
                            preferred_element_type=jnp.float32)
                    + b2_ref[...][None, :], 0.0)
    h = jnp.dot(h, w3_ref[...], preferred_element_type=jnp.float32) \
        + b3_ref[...][None, :]
    nrm = jnp.sqrt(jnp.sum(h * h, axis=1, keepdims=True))
    out_ref[...] = h / (nrm + 1e-8)


def _tc_mlp(img, w0, b0, w1, b1, w2, b2, w3p, b3p):
    B = img.shape[0]
    bm = 256
    grid = (B // bm,)
    full = lambda *s: pl.BlockSpec(s, lambda i: tuple(0 for _ in s))
    return pl.pallas_call(
        _mlp_body,
        grid=grid,
        in_specs=[
            pl.BlockSpec((bm, 512), lambda i: (i, 0)),
            full(512, 768), full(768,),
            full(768, 1024), full(1024,),
            full(1024, 1200), full(1200,),
            full(1200, DP), full(DP,),
        ],
        out_specs=pl.BlockSpec((bm, DP), lambda i: (i, 0)),
        out_shape=jax.ShapeDtypeStruct((B, DP), jnp.float32),
    )(img, w0, b0, w1, b1, w2, b2, w3p, b3p)


def _score_body(a_ref, b_ref, degp_ref, bg2_ref, imgf_ref, out_ref):
    dinv = _dinv_of(degp_ref[...])
    pair = jnp.concatenate(
        [a_ref[p] + b_ref[p] for p in range(NPASS)], axis=1
    ) * dinv[:, None] + bg2_ref[...][None, :]
    out_ref[...] = lax.dot_general(
        imgf_ref[...], pair, (((1,), (1,)), ((), ())),
        preferred_element_type=jnp.float32)


def _tc_score(o2a, o2b, degp_pair, bg2p, imgf):
    bn = 1280
    npair = o2a.shape[1]
    grid = (npair // bn,)
    B = imgf.shape[0]
    return pl.pallas_call(
        _score_body,
        grid=grid,
        in_specs=[
            pl.BlockSpec((NPASS, bn, DC), lambda j: (0, j, 0)),
            pl.BlockSpec((NPASS, bn, DC), lambda j: (0, j, 0)),
            pl.BlockSpec((2, bn, 16), lambda j: (0, j, 0)),
            pl.BlockSpec((DP,), lambda j: (0,)),
            pl.BlockSpec((B, DP), lambda j: (0, 0)),
        ],
        out_specs=pl.BlockSpec((B, bn), lambda j: (0, j)),
        out_shape=jax.ShapeDtypeStruct((B, npair), jnp.float32),
    )(o2a, o2b, degp_pair, bg2p, imgf)


@jax.jit
def kernel(img, embeddings, edge_index, W0, b0, W1, b1, W2, b2, W3, b3,
           Wg1, bg1, Wg2, bg2):
    f32 = jnp.float32
    row = edge_index[0].astype(jnp.int32)
    col = edge_index[1].astype(jnp.int32)
    E = row.shape[0]
    row = jnp.pad(row, (0, EP - E), constant_values=N_NODES)
    col = jnp.pad(col, (0, EP - E), constant_values=N_NODES)
    rowb = row.reshape(NT, NB_E, EB)
    colb = col.reshape(NT, NB_E, EB)
    embp = jnp.pad(embeddings.astype(f32),
                   ((0, NP - N_NODES), (0, DP - embeddings.shape[1])))
    wg1p = jnp.pad(Wg1.astype(f32), ((0, DP - Wg1.shape[0]), (0, 0)))
    wg2p = jnp.pad(Wg2.astype(f32), ((0, 0), (0, DP - Wg2.shape[1])))
    bg2p = jnp.pad(bg2.astype(f32), (0, DP - bg2.shape[0]))
    w3p = jnp.pad(W3.astype(f32), ((0, 0), (0, DP - W3.shape[1])))
    b3p = jnp.pad(b3.astype(f32), (0, DP - b3.shape[0]))
    ones16 = jnp.ones((EB, 16), f32)
    zeros16 = jnp.zeros((128, 16), f32)
    zeros64 = jnp.zeros((128, DC), f32)

    degp = _sc_deg(rowb, ones16, zeros16)
    xs = _tc_prescale(degp, embp)
    xs_t = jnp.transpose(xs.reshape(NP, NPASS, DC), (1, 0, 2))

    o1 = _sc_spmm(xs_t, rowb, colb, zeros64)
    g2s = _tc_gcn_mm(o1[0], o1[1], degp, wg1p, bg1.astype(f32), wg2p)
    g2s_t = jnp.transpose(g2s.reshape(NP, NPASS, DC), (1, 0, 2))

    o2 = _sc_spmm(g2s_t, rowb, colb, zeros64)

    imgf = _tc_mlp(img.astype(f32), W0.astype(f32), b0.astype(f32),
                   W1.astype(f32), b1.astype(f32), W2.astype(f32),
                   b2.astype(f32), w3p, b3p)
    npair = N_NODES - DISP
    npair_p = 30720
    pad = npair_p - npair
    o2a = jnp.pad(o2[0, :, DISP:N_NODES, :], ((0, 0), (0, pad), (0, 0)))
    o2b = jnp.pad(o2[1, :, DISP:N_NODES, :], ((0, 0), (0, pad), (0, 0)))
    degp_pair = jnp.pad(degp[:, DISP:N_NODES, :],
                        ((0, 0), (0, pad), (0, 0)))
    score = _tc_score(o2a, o2b, degp_pair, bg2p, imgf)
    return score[:, :npair]


# speedup vs baseline: 8.3061x; 1.2961x over previous
import functools

import jax
import jax.numpy as jnp
from jax import lax
from jax.experimental import pallas as pl
from jax.experimental.pallas import tpu as pltpu
from jax.experimental.pallas import tpu_sc as plsc

DISP = 1100
N_NODES = 31100
NP = 31104
DP = 320
DC = 32
NPASS = DP // DC
NT = 32
EB = 128
NB_E = 52
EP = NT * NB_E * EB
ROWS_PT = NP // 16

_MESH = plsc.VectorSubcoreMesh(core_axis_name="c", subcore_axis_name="s")
_SC_PARAMS = pltpu.CompilerParams(use_tc_tiling_on_sc=False)


def _sc_deg_body(rowb_hbm, ones_hbm, zeros_hbm, out_hbm,
                 rowbuf, onesb, zbuf, acc):
    c = lax.axis_index("c")
    s = lax.axis_index("s")
    wid = c * 16 + s
    pltpu.sync_copy(rowb_hbm.at[wid], rowbuf)
    pltpu.sync_copy(ones_hbm, onesb)
    pltpu.sync_copy(zeros_hbm, zbuf)
    r0 = s * ROWS_PT
    for off in range(0, ROWS_PT - 128 + 1, 128):
        pltpu.sync_copy(zbuf, acc.at[pl.ds(r0 + off, 128)])
    rem = ROWS_PT % 128
    if rem:
        pltpu.sync_copy(zbuf.at[pl.ds(0, rem)],
                        acc.at[pl.ds(r0 + ROWS_PT - rem, rem)])
    plsc.subcore_barrier()
    for j in range(NB_E):
        pltpu.sync_copy(onesb, acc.at[rowbuf.at[j]], add=True)
    plsc.subcore_barrier()
    for off in range(0, ROWS_PT - 128 + 1, 128):
        pltpu.sync_copy(acc.at[pl.ds(r0 + off, 128)],
                        out_hbm.at[c, pl.ds(r0 + off, 128)])
    if rem:
        pltpu.sync_copy(acc.at[pl.ds(r0 + ROWS_PT - rem, rem)],
                        out_hbm.at[c, pl.ds(r0 + ROWS_PT - rem, rem)])


_sc_deg = pl.kernel(
    _sc_deg_body,
    out_type=jax.ShapeDtypeStruct((2, NP, 16), jnp.float32),
    mesh=_MESH,
    scratch_types=[
        pltpu.VMEM((NB_E, EB), jnp.int32),
        pltpu.VMEM((EB, 16), jnp.float32),
        pltpu.VMEM((128, 16), jnp.float32),
        pltpu.VMEM_SHARED((NP, 16), jnp.float32),
    ],
    compiler_params=_SC_PARAMS,
)


_NBUF = 4


def _sc_spmm_body(xs_hbm, rowb_hbm, colb_hbm, zeros_hbm, out_hbm,
                  rowbuf, colbuf, gbufs, zbuf, acc, gsems, ssems):
    c = lax.axis_index("c")
    s = lax.axis_index("s")
    wid = c * 16 + s
    pltpu.sync_copy(rowb_hbm.at[wid], rowbuf)
    pltpu.sync_copy(colb_hbm.at[wid], colbuf)
    pltpu.sync_copy(zeros_hbm, zbuf)
    r0 = s * ROWS_PT
    rem = ROWS_PT % 128

    def one_pass(p, carry):
        for off in range(0, ROWS_PT - 128 + 1, 128):
            pltpu.sync_copy(zbuf, acc.at[pl.ds(r0 + off, 128)])
        if rem:
            pltpu.sync_copy(zbuf.at[pl.ds(0, rem)],
                            acc.at[pl.ds(r0 + ROWS_PT - rem, rem)])
        plsc.subcore_barrier()
        xs_p = xs_hbm.at[p]
        gd = [None] * _NBUF
        sd = [None] * _NBUF
        for j in range(NB_E):
            b = j % _NBUF
            if j >= _NBUF:
                sd[b].wait()
            gd[b] = pltpu.async_copy(
                xs_p.at[colbuf.at[j]], gbufs.at[b], gsems.at[b])
            jp = j - 1
            if jp >= 0:
                bp = jp % _NBUF
                gd[bp].wait()
                sd[bp] = pltpu.async_copy(
                    gbufs.at[bp], acc.at[rowbuf.at[jp]], ssems.at[bp],
                    add=True)
        b = (NB_E - 1) % _NBUF
        gd[b].wait()
        sd[b] = pltpu.async_copy(
            gbufs.at[b], acc.at[rowbuf.at[NB_E - 1]], ssems.at[b], add=True)
        for b in range(_NBUF):
            sd[b].wait()
        plsc.subcore_barrier()
        for off in range(0, ROWS_PT - 128 + 1, 128):
            pltpu.sync_copy(
                acc.at[pl.ds(r0 + off, 128)],
                out_hbm.at[c, pl.ds(r0 + off, 128), pl.ds(p * DC, DC)])
        if rem:
            pltpu.sync_copy(
                acc.at[pl.ds(r0 + ROWS_PT - rem, rem)],
                out_hbm.at[c, pl.ds(r0 + ROWS_PT - rem, rem),
                           pl.ds(p * DC, DC)])
        plsc.subcore_barrier()
        return carry

    lax.fori_loop(0, NPASS, one_pass, 0)


_sc_spmm = pl.kernel(
    _sc_spmm_body,
    out_type=jax.ShapeDtypeStruct((2, NP, DP), jnp.float32),
    mesh=_MESH,
    scratch_types=[
        pltpu.VMEM((NB_E, EB), jnp.int32),
        pltpu.VMEM((NB_E, EB), jnp.int32),
        pltpu.VMEM((_NBUF, EB, DC), jnp.float32),
        pltpu.VMEM((128, DC), jnp.float32),
        pltpu.VMEM_SHARED((NP, DC), jnp.float32),
        pltpu.SemaphoreType.DMA((_NBUF,)),
        pltpu.SemaphoreType.DMA((_NBUF,)),
    ],
    compiler_params=_SC_PARAMS,
)


def _dinv_of(degp_block):
    deg = jnp.sum(degp_block, axis=(0, 2)) * (1.0 / 16.0)
    return 1.0 / jnp.sqrt(jnp.maximum(deg, 1.0))


def _prescale_body(degp_ref, emb_ref, xs_ref):
    dinv = _dinv_of(degp_ref[...])
    xs_ref[...] = emb_ref[...] * dinv[:, None]


def _tc_prescale(degp, embp):
    bm = 3888
    grid = (NP // bm,)
    return pl.pallas_call(
        _prescale_body,
        grid=grid,
        in_specs=[
            pl.BlockSpec((2, bm, 16), lambda i: (0, i, 0)),
            pl.BlockSpec((bm, DP), lambda i: (i, 0)),
        ],
        out_specs=pl.BlockSpec((bm, DP), lambda i: (i, 0)),
        out_shape=jax.ShapeDtypeStruct((NP, DP), jnp.float32),
    )(degp, embp)


def _gcn_mm_body(p0_ref, p1_ref, degp_ref, wg1_ref, bg1_ref, wg2_ref,
                 out_ref):
    dinv = _dinv_of(degp_ref[...])
    h1 = (p0_ref[...] + p1_ref[...]) * dinv[:, None]
    acc = jnp.dot(h1, wg1_ref[...], preferred_element_type=jnp.float32)
    g = jnp.maximum(acc + bg1_ref[...][None, :], 0.0)
    g2 = jnp.dot(g, wg2_ref[...], preferred_element_type=jnp.float32)
    out_ref[...] = g2 * dinv[:, None]


def _tc_gcn_mm(p0, p1, degp, wg1p, bg1, wg2p):
    bm = 432
    grid = (NP // bm,)
    return pl.pallas_call(
        _gcn_mm_body,
        grid=grid,
        in_specs=[
            pl.BlockSpec((bm, DP), lambda i: (i, 0)),
            pl.BlockSpec((bm, DP), lambda i: (i, 0)),
            pl.BlockSpec((2, bm, 16), lambda i: (0, i, 0)),
            pl.BlockSpec((DP, 4096), lambda i: (0, 0)),
            pl.BlockSpec((4096,), lambda i: (0,)),
            pl.BlockSpec((4096, DP), lambda i: (0, 0)),
        ],
        out_specs=pl.BlockSpec((bm, DP), lambda i: (i, 0)),
        out_shape=jax.ShapeDtypeStruct((NP, DP), jnp.float32),
    )(p0, p1, degp, wg1p, bg1, wg2p)


def _mlp_body(img_ref, w0_ref, b0_ref, w1_ref, b1_ref, w2_ref, b2_ref,
              w3_ref, b3_ref, out_ref):
    h = jnp.maximum(jnp.dot(img_ref[...], w0_ref[...],
                            preferred_element_type=jnp.float32)
                    + b0_ref[...][None, :], 0.0)
    h = jnp.maximum(jnp.dot(h, w1_ref[...],
                            preferred_element_type=jnp.float32)
                    + b1_ref[...][None, :], 0.0)
    h = jnp.maximum(jnp.dot(h, w2_ref[...],
                            preferred_element_type=jnp.float32)
                    + b2_ref[...][None, :], 0.0)
    h = jnp.dot(h, w3_ref[...], preferred_element_type=jnp.float32) \
        + b3_ref[...][None, :]
    nrm = jnp.sqrt(jnp.sum(h * h, axis=1, keepdims=True))
    out_ref[...] = h / (nrm + 1e-8)


def _tc_mlp(img, w0, b0, w1, b1, w2, b2, w3p, b3p):
    B = img.shape[0]
    bm = 256
    grid = (B // bm,)
    full = lambda *s: pl.BlockSpec(s, lambda i: tuple(0 for _ in s))
    return pl.pallas_call(
        _mlp_body,
        grid=grid,
        in_specs=[
            pl.BlockSpec((bm, 512), lambda i: (i, 0)),
            full(512, 768), full(768,),
            full(768, 1024), full(1024,),
            full(1024, 1200), full(1200,),
            full(1200, DP), full(DP,),
        ],
        out_specs=pl.BlockSpec((bm, DP), lambda i: (i, 0)),
        out_shape=jax.ShapeDtypeStruct((B, DP), jnp.float32),
    )(img, w0, b0, w1, b1, w2, b2, w3p, b3p)


def _score_body(a_ref, b_ref, degp_ref, bg2_ref, imgf_ref, out_ref):
    dinv = _dinv_of(degp_ref[...])
    pair = (a_ref[...] + b_ref[...]) * dinv[:, None] \
        + bg2_ref[...][None, :]
    out_ref[...] = lax.dot_general(
        imgf_ref[...], pair, (((1,), (1,)), ((), ())),
        preferred_element_type=jnp.float32)


def _tc_score(o2a, o2b, degp_pair, bg2p, imgf):
    bn = 1280
    npair = o2a.shape[0]
    grid = (npair // bn,)
    B = imgf.shape[0]
    return pl.pallas_call(
        _score_body,
        grid=grid,
        in_specs=[
            pl.BlockSpec((bn, DP), lambda j: (j, 0)),
            pl.BlockSpec((bn, DP), lambda j: (j, 0)),
            pl.BlockSpec((2, bn, 16), lambda j: (0, j, 0)),
            pl.BlockSpec((DP,), lambda j: (0,)),
            pl.BlockSpec((B, DP), lambda j: (0, 0)),
        ],
        out_specs=pl.BlockSpec((B, bn), lambda j: (0, j)),
        out_shape=jax.ShapeDtypeStruct((B, npair), jnp.float32),
    )(o2a, o2b, degp_pair, bg2p, imgf)


@jax.jit
def kernel(img, embeddings, edge_index, W0, b0, W1, b1, W2, b2, W3, b3,
           Wg1, bg1, Wg2, bg2):
    f32 = jnp.float32
    row = edge_index[0].astype(jnp.int32)
    col = edge_index[1].astype(jnp.int32)
    E = row.shape[0]
    row = jnp.pad(row, (0, EP - E), constant_values=N_NODES)
    col = jnp.pad(col, (0, EP - E), constant_values=N_NODES)
    rowb = row.reshape(NT, NB_E, EB)
    colb = col.reshape(NT, NB_E, EB)
    embp = jnp.pad(embeddings.astype(f32),
                   ((0, NP - N_NODES), (0, DP - embeddings.shape[1])))
    wg1p = jnp.pad(Wg1.astype(f32), ((0, DP - Wg1.shape[0]), (0, 0)))
    wg2p = jnp.pad(Wg2.astype(f32), ((0, 0), (0, DP - Wg2.shape[1])))
    bg2p = jnp.pad(bg2.astype(f32), (0, DP - bg2.shape[0]))
    w3p = jnp.pad(W3.astype(f32), ((0, 0), (0, DP - W3.shape[1])))
    b3p = jnp.pad(b3.astype(f32), (0, DP - b3.shape[0]))
    ones16 = jnp.ones((EB, 16), f32)
    zeros16 = jnp.zeros((128, 16), f32)
    zeros64 = jnp.zeros((128, DC), f32)

    degp = _sc_deg(rowb, ones16, zeros16)
    xs = _tc_prescale(degp, embp)
    xs_t = xs.reshape(NP, NPASS, DC).transpose(1, 0, 2)

    o1 = _sc_spmm(xs_t, rowb, colb, zeros64)
    g2s = _tc_gcn_mm(o1[0], o1[1], degp, wg1p, bg1.astype(f32), wg2p)
    g2s_t = g2s.reshape(NP, NPASS, DC).transpose(1, 0, 2)

    o2 = _sc_spmm(g2s_t, rowb, colb, zeros64)

    imgf = _tc_mlp(img.astype(f32), W0.astype(f32), b0.astype(f32),
                   W1.astype(f32), b1.astype(f32), W2.astype(f32),
                   b2.astype(f32), w3p, b3p)
    npair = N_NODES - DISP
    npair_p = 30720
    pad = npair_p - npair
    o2a = jnp.pad(o2[0, DISP:N_NODES, :], ((0, pad), (0, 0)))
    o2b = jnp.pad(o2[1, DISP:N_NODES, :], ((0, pad), (0, 0)))
    degp_pair = jnp.pad(degp[:, DISP:N_NODES, :],
                        ((0, 0), (0, pad), (0, 0)))
    score = _tc_score(o2a, o2b, degp_pair, bg2p, imgf)
    return score[:, :npair]


# speedup vs baseline: 8.6264x; 1.0386x over previous
import functools

import jax
import jax.numpy as jnp
from jax import lax
from jax.experimental import pallas as pl
from jax.experimental.pallas import tpu as pltpu
from jax.experimental.pallas import tpu_sc as plsc

DISP = 1100
N_NODES = 31100
NP = 31104
DP = 320
DC = 40
NPASS = DP // DC
NT = 32
EB = 128
NB_E = 52
EP = NT * NB_E * EB
ROWS_PT = NP // 16

_MESH = plsc.VectorSubcoreMesh(core_axis_name="c", subcore_axis_name="s")
_SC_PARAMS = pltpu.CompilerParams(use_tc_tiling_on_sc=False)


def _sc_deg_body(rowb_hbm, ones_hbm, zeros_hbm, out_hbm,
                 rowbuf, onesb, zbuf, acc):
    c = lax.axis_index("c")
    s = lax.axis_index("s")
    wid = c * 16 + s
    pltpu.sync_copy(rowb_hbm.at[wid], rowbuf)
    pltpu.sync_copy(ones_hbm, onesb)
    pltpu.sync_copy(zeros_hbm, zbuf)
    r0 = s * ROWS_PT
    for off in range(0, ROWS_PT - 128 + 1, 128):
        pltpu.sync_copy(zbuf, acc.at[pl.ds(r0 + off, 128)])
    rem = ROWS_PT % 128
    if rem:
        pltpu.sync_copy(zbuf.at[pl.ds(0, rem)],
                        acc.at[pl.ds(r0 + ROWS_PT - rem, rem)])
    plsc.subcore_barrier()
    for j in range(NB_E):
        pltpu.sync_copy(onesb, acc.at[rowbuf.at[j]], add=True)
    plsc.subcore_barrier()
    for off in range(0, ROWS_PT - 128 + 1, 128):
        pltpu.sync_copy(acc.at[pl.ds(r0 + off, 128)],
                        out_hbm.at[c, pl.ds(r0 + off, 128)])
    if rem:
        pltpu.sync_copy(acc.at[pl.ds(r0 + ROWS_PT - rem, rem)],
                        out_hbm.at[c, pl.ds(r0 + ROWS_PT - rem, rem)])


_sc_deg = pl.kernel(
    _sc_deg_body,
    out_type=jax.ShapeDtypeStruct((2, NP, 16), jnp.float32),
    mesh=_MESH,
    scratch_types=[
        pltpu.VMEM((NB_E, EB), jnp.int32),
        pltpu.VMEM((EB, 16), jnp.float32),
        pltpu.VMEM((128, 16), jnp.float32),
        pltpu.VMEM_SHARED((NP, 16), jnp.float32),
    ],
    compiler_params=_SC_PARAMS,
)


_NBUF = 4


def _sc_spmm_body(xs_hbm, rowb_hbm, colb_hbm, zeros_hbm, out_hbm,
                  rowbuf, colbuf, gbufs, zbuf, acc, gsems, ssems):
    c = lax.axis_index("c")
    s = lax.axis_index("s")
    wid = c * 16 + s
    pltpu.sync_copy(rowb_hbm.at[wid], rowbuf)
    pltpu.sync_copy(colb_hbm.at[wid], colbuf)
    pltpu.sync_copy(zeros_hbm, zbuf)
    r0 = s * ROWS_PT
    rem = ROWS_PT % 128

    def one_pass(p, carry):
        for off in range(0, ROWS_PT - 128 + 1, 128):
            pltpu.sync_copy(zbuf, acc.at[pl.ds(r0 + off, 128)])
        if rem:
            pltpu.sync_copy(zbuf.at[pl.ds(0, rem)],
                            acc.at[pl.ds(r0 + ROWS_PT - rem, rem)])
        plsc.subcore_barrier()
        xs_p = xs_hbm.at[p]
        gd = [None] * _NBUF
        sd = [None] * _NBUF
        for j in range(NB_E):
            b = j % _NBUF
            if j >= _NBUF:
                sd[b].wait()
            gd[b] = pltpu.async_copy(
                xs_p.at[colbuf.at[j]], gbufs.at[b], gsems.at[b])
            jp = j - 1
            if jp >= 0:
                bp = jp % _NBUF
                gd[bp].wait()
                sd[bp] = pltpu.async_copy(
                    gbufs.at[bp], acc.at[rowbuf.at[jp]], ssems.at[bp],
                    add=True)
        b = (NB_E - 1) % _NBUF
        gd[b].wait()
        sd[b] = pltpu.async_copy(
            gbufs.at[b], acc.at[rowbuf.at[NB_E - 1]], ssems.at[b], add=True)
        for b in range(_NBUF):
            sd[b].wait()
        plsc.subcore_barrier()
        for off in range(0, ROWS_PT - 128 + 1, 128):
            pltpu.sync_copy(
                acc.at[pl.ds(r0 + off, 128)],
                out_hbm.at[c, pl.ds(r0 + off, 128), pl.ds(p * DC, DC)])
        if rem:
            pltpu.sync_copy(
                acc.at[pl.ds(r0 + ROWS_PT - rem, rem)],
                out_hbm.at[c, pl.ds(r0 + ROWS_PT - rem, rem),
                           pl.ds(p * DC, DC)])
        plsc.subcore_barrier()
        return carry

    lax.fori_loop(0, NPASS, one_pass, 0)


_sc_spmm = pl.kernel(
    _sc_spmm_body,
    out_type=jax.ShapeDtypeStruct((2, NP, DP), jnp.float32),
    mesh=_MESH,
    scratch_types=[
        pltpu.VMEM((NB_E, EB), jnp.int32),
        pltpu.VMEM((NB_E, EB), jnp.int32),
        pltpu.VMEM((_NBUF, EB, DC), jnp.float32),
        pltpu.VMEM((128, DC), jnp.float32),
        pltpu.VMEM_SHARED((NP, DC), jnp.float32),
        pltpu.SemaphoreType.DMA((_NBUF,)),
        pltpu.SemaphoreType.DMA((_NBUF,)),
    ],
    compiler_params=_SC_PARAMS,
)


def _dinv_of(degp_block):
    deg = jnp.sum(degp_block, axis=(0, 2)) * (1.0 / 16.0)
    return 1.0 / jnp.sqrt(jnp.maximum(deg, 1.0))


def _prescale_body(degp_ref, emb_ref, xs_ref):
    dinv = _dinv_of(degp_ref[...])
    xs_ref[...] = emb_ref[...] * dinv[:, None]


def _tc_prescale(degp, embp):
    bm = 3888
    grid = (NP // bm,)
    return pl.pallas_call(
        _prescale_body,
        grid=grid,
        in_specs=[
            pl.BlockSpec((2, bm, 16), lambda i: (0, i, 0)),
            pl.BlockSpec((bm, DP), lambda i: (i, 0)),
        ],
        out_specs=pl.BlockSpec((bm, DP), lambda i: (i, 0)),
        out_shape=jax.ShapeDtypeStruct((NP, DP), jnp.float32),
    )(degp, embp)


def _gcn_mm_body(p0_ref, p1_ref, degp_ref, wg1_ref, bg1_ref, wg2_ref,
                 out_ref):
    dinv = _dinv_of(degp_ref[...])
    h1 = (p0_ref[...] + p1_ref[...]) * dinv[:, None]
    acc = jnp.dot(h1, wg1_ref[...], preferred_element_type=jnp.float32)
    g = jnp.maximum(acc + bg1_ref[...][None, :], 0.0)
    g2 = jnp.dot(g, wg2_ref[...], preferred_element_type=jnp.float32)
    out_ref[...] = g2 * dinv[:, None]


def _tc_gcn_mm(p0, p1, degp, wg1p, bg1, wg2p):
    bm = 432
    grid = (NP // bm,)
    return pl.pallas_call(
        _gcn_mm_body,
        grid=grid,
        in_specs=[
            pl.BlockSpec((bm, DP), lambda i: (i, 0)),
            pl.BlockSpec((bm, DP), lambda i: (i, 0)),
            pl.BlockSpec((2, bm, 16), lambda i: (0, i, 0)),
            pl.BlockSpec((DP, 4096), lambda i: (0, 0)),
            pl.BlockSpec((4096,), lambda i: (0,)),
            pl.BlockSpec((4096, DP), lambda i: (0, 0)),
        ],
        out_specs=pl.BlockSpec((bm, DP), lambda i: (i, 0)),
        out_shape=jax.ShapeDtypeStruct((NP, DP), jnp.float32),
    )(p0, p1, degp, wg1p, bg1, wg2p)


def _mlp_body(img_ref, w0_ref, b0_ref, w1_ref, b1_ref, w2_ref, b2_ref,
              w3_ref, b3_ref, out_ref):
    h = jnp.maximum(jnp.dot(img_ref[...], w0_ref[...],
                            preferred_element_type=jnp.float32)
                    + b0_ref[...][None, :], 0.0)
    h = jnp.maximum(jnp.dot(h, w1_ref[...],
                            preferred_element_type=jnp.float32)
                    + b1_ref[...][None, :], 0.0)
    h = jnp.maximum(jnp.dot(h, w2_ref[...],
                            preferred_element_type=jnp.float32)
                    + b2_ref[...][None, :], 0.0)
    h = jnp.dot(h, w3_ref[...], preferred_element_type=jnp.float32) \
        + b3_ref[...][None, :]
    nrm = jnp.sqrt(jnp.sum(h * h, axis=1, keepdims=True))
    out_ref[...] = h / (nrm + 1e-8)


def _tc_mlp(img, w0, b0, w1, b1, w2, b2, w3p, b3p):
    B = img.shape[0]
    bm = 256
    grid = (B // bm,)
    full = lambda *s: pl.BlockSpec(s, lambda i: tuple(0 for _ in s))
    return pl.pallas_call(
        _mlp_body,
        grid=grid,
        in_specs=[
            pl.BlockSpec((bm, 512), lambda i: (i, 0)),
            full(512, 768), full(768,),
            full(768, 1024), full(1024,),
            full(1024, 1200), full(1200,),
            full(1200, DP), full(DP,),
        ],
        out_specs=pl.BlockSpec((bm, DP), lambda i: (i, 0)),
        out_shape=jax.ShapeDtypeStruct((B, DP), jnp.float32),
    )(img, w0, b0, w1, b1, w2, b2, w3p, b3p)


def _score_body(a_ref, b_ref, degp_ref, bg2_ref, imgf_ref, out_ref):
    dinv = _dinv_of(degp_ref[...])
    pair = (a_ref[...] + b_ref[...]) * dinv[:, None] \
        + bg2_ref[...][None, :]
    out_ref[...] = lax.dot_general(
        imgf_ref[...], pair, (((1,), (1,)), ((), ())),
        preferred_element_type=jnp.float32)


def _tc_score(o2a, o2b, degp_pair, bg2p, imgf):
    bn = 1280
    npair = o2a.shape[0]
    grid = (npair // bn,)
    B = imgf.shape[0]
    return pl.pallas_call(
        _score_body,
        grid=grid,
        in_specs=[
            pl.BlockSpec((bn, DP), lambda j: (j, 0)),
            pl.BlockSpec((bn, DP), lambda j: (j, 0)),
            pl.BlockSpec((2, bn, 16), lambda j: (0, j, 0)),
            pl.BlockSpec((DP,), lambda j: (0,)),
            pl.BlockSpec((B, DP), lambda j: (0, 0)),
        ],
        out_specs=pl.BlockSpec((B, bn), lambda j: (0, j)),
        out_shape=jax.ShapeDtypeStruct((B, npair), jnp.float32),
    )(o2a, o2b, degp_pair, bg2p, imgf)


@jax.jit
def kernel(img, embeddings, edge_index, W0, b0, W1, b1, W2, b2, W3, b3,
           Wg1, bg1, Wg2, bg2):
    f32 = jnp.float32
    row = edge_index[0].astype(jnp.int32)
    col = edge_index[1].astype(jnp.int32)
    E = row.shape[0]
    row = jnp.pad(row, (0, EP - E), constant_values=N_NODES)
    col = jnp.pad(col, (0, EP - E), constant_values=N_NODES)
    rowb = row.reshape(NT, NB_E, EB)
    colb = col.reshape(NT, NB_E, EB)
    embp = jnp.pad(embeddings.astype(f32),
                   ((0, NP - N_NODES), (0, DP - embeddings.shape[1])))
    wg1p = jnp.pad(Wg1.astype(f32), ((0, DP - Wg1.shape[0]), (0, 0)))
    wg2p = jnp.pad(Wg2.astype(f32), ((0, 0), (0, DP - Wg2.shape[1])))
    bg2p = jnp.pad(bg2.astype(f32), (0, DP - bg2.shape[0]))
    w3p = jnp.pad(W3.astype(f32), ((0, 0), (0, DP - W3.shape[1])))
    b3p = jnp.pad(b3.astype(f32), (0, DP - b3.shape[0]))
    ones16 = jnp.ones((EB, 16), f32)
    zeros16 = jnp.zeros((128, 16), f32)
    zeros64 = jnp.zeros((128, DC), f32)

    degp = _sc_deg(rowb, ones16, zeros16)
    xs = _tc_prescale(degp, embp)
    xs_t = xs.reshape(NP, NPASS, DC).transpose(1, 0, 2)

    o1 = _sc_spmm(xs_t, rowb, colb, zeros64)
    g2s = _tc_gcn_mm(o1[0], o1[1], degp, wg1p, bg1.astype(f32), wg2p)
    g2s_t = g2s.reshape(NP, NPASS, DC).transpose(1, 0, 2)

    o2 = _sc_spmm(g2s_t, rowb, colb, zeros64)

    imgf = _tc_mlp(img.astype(f32), W0.astype(f32), b0.astype(f32),
                   W1.astype(f32), b1.astype(f32), W2.astype(f32),
                   b2.astype(f32), w3p, b3p)
    npair = N_NODES - DISP
    npair_p = 30720
    pad = npair_p - npair
    o2a = jnp.pad(o2[0, DISP:N_NODES, :], ((0, pad), (0, 0)))
    o2b = jnp.pad(o2[1, DISP:N_NODES, :], ((0, pad), (0, 0)))
    degp_pair = jnp.pad(degp[:, DISP:N_NODES, :],
                        ((0, 0), (0, pad), (0, 0)))
    score = _tc_score(o2a, o2b, degp_pair, bg2p, imgf)
    return score[:, :npair]
